# Initial kernel scaffold; baseline (speedup 1.0000x reference)
import jax
import jax.numpy as jnp
from jax.experimental import pallas as pl


def kernel(x, edge_index, W1, b1, W2, b2, W3, b3, Wm1, bm1, Wm2, bm2):
    raise NotImplementedError("write your pallas kernel here")


# speedup vs baseline: 17.6274x; 17.6274x over previous
import functools

import jax
import jax.numpy as jnp
from jax import lax
from jax.experimental import pallas as pl
from jax.experimental.pallas import tpu as pltpu
from jax.experimental.pallas import tpu_sc as plsc

N_NODES = 10000
N_EDGES = 320000
IN_CH = 128
HID = 64
OUT_CH = 64

NC, NS = 2, 16
NW = NC * NS
EPW = N_EDGES // NW
CHUNK = 80
NCHUNK = EPW // CHUNK
DEGW = 16
ROW_BLK = 80
NROWBLK = N_NODES // ROW_BLK

_MESH = plsc.VectorSubcoreMesh(core_axis_name="c", subcore_axis_name="s")


def _worker_ids():
    c = lax.axis_index("c")
    s = lax.axis_index("s")
    return c, s, c * NS + s


def _deg_body(dst_hbm, ones_hbm, zeros_hbm, out_hbm, dstv, onesv, zerosv, acc, sem):
    c, s, w = _worker_ids()
    pltpu.sync_copy(zeros_hbm, zerosv)

    @pl.loop(s, NROWBLK, step=NS)
    def _zero(k):
        pltpu.sync_copy(zerosv, acc.at[pl.ds(k * ROW_BLK, ROW_BLK)])

    plsc.subcore_barrier()
    pltpu.sync_copy(ones_hbm, onesv)
    pltpu.sync_copy(dst_hbm.at[w], dstv)

    @pl.loop(0, NCHUNK)
    def _edges(j):
        pltpu.sync_copy(onesv, acc.at[dstv.at[j]], add=True)

    plsc.subcore_barrier()

    @pl.loop(s, NROWBLK, step=NS)
    def _out(k):
        pltpu.sync_copy(acc.at[pl.ds(k * ROW_BLK, ROW_BLK)],
                        out_hbm.at[c, pl.ds(k * ROW_BLK, ROW_BLK)])


_sc_deg = pl.kernel(
    _deg_body,
    out_type=jax.ShapeDtypeStruct((NC, N_NODES, DEGW), jnp.float32),
    mesh=_MESH,
    scratch_types=[
        pltpu.VMEM((NCHUNK, CHUNK), jnp.int32),
        pltpu.VMEM((CHUNK, DEGW), jnp.float32),
        pltpu.VMEM((ROW_BLK, DEGW), jnp.float32),
        pltpu.VMEM_SHARED((N_NODES, DEGW), jnp.float32),
        pltpu.SemaphoreType.DMA,
    ],
    compiler_params=pltpu.CompilerParams(use_tc_tiling_on_sc=False),
)


def _scatter_body(g_hbm, src_hbm, dst_hbm, zeros_hbm, out_hbm,
                  srcv, dstv, rows, zerosv, acc, sem):
    c, s, w = _worker_ids()
    pltpu.sync_copy(zeros_hbm, zerosv)

    @pl.loop(s, NROWBLK, step=NS)
    def _zero(k):
        pltpu.sync_copy(zerosv, acc.at[pl.ds(k * ROW_BLK, ROW_BLK)])

    plsc.subcore_barrier()
    pltpu.sync_copy(src_hbm.at[w], srcv)
    pltpu.sync_copy(dst_hbm.at[w], dstv)

    @pl.loop(0, NCHUNK)
    def _edges(j):
        pltpu.async_copy(g_hbm.at[srcv.at[j]], rows, sem).wait()
        pltpu.sync_copy(rows, acc.at[dstv.at[j]], add=True)

    plsc.subcore_barrier()

    @pl.loop(s, NROWBLK, step=NS)
    def _out(k):
        pltpu.sync_copy(acc.at[pl.ds(k * ROW_BLK, ROW_BLK)],
                        out_hbm.at[c, pl.ds(k * ROW_BLK, ROW_BLK)])


_sc_scatter = pl.kernel(
    _scatter_body,
    out_type=jax.ShapeDtypeStruct((NC, N_NODES, HID), jnp.float32),
    mesh=_MESH,
    scratch_types=[
        pltpu.VMEM((NCHUNK, CHUNK), jnp.int32),
        pltpu.VMEM((NCHUNK, CHUNK), jnp.int32),
        pltpu.VMEM((CHUNK, HID), jnp.float32),
        pltpu.VMEM((ROW_BLK, HID), jnp.float32),
        pltpu.VMEM_SHARED((N_NODES, HID), jnp.float32),
        pltpu.SemaphoreType.DMA,
    ],
    compiler_params=pltpu.CompilerParams(use_tc_tiling_on_sc=False),
)


RB = 1000
GRID = (N_NODES // RB,)


def _dinv(da_ref, db_ref):
    return lax.rsqrt(1.0 + da_ref[:, 0:1] + db_ref[:, 0:1])


def _tc0_body(x_ref, w_ref, da_ref, db_ref, g_ref):
    g_ref[...] = _dinv(da_ref, db_ref) * jnp.dot(
        x_ref[...], w_ref[...], preferred_element_type=jnp.float32)


def _tc_mid_body(aa_ref, ab_ref, g_ref, da_ref, db_ref, b_ref, w_ref,
                 h_ref, g2_ref):
    dinv = _dinv(da_ref, db_ref)
    h = jnp.maximum(dinv * (aa_ref[...] + ab_ref[...] + g_ref[...]) + b_ref[...], 0.0)
    h_ref[...] = h
    g2_ref[...] = dinv * jnp.dot(h, w_ref[...], preferred_element_type=jnp.float32)


def _tc_fin_body(aa_ref, ab_ref, g_ref, da_ref, db_ref, b_ref,
                 x_ref, h1_ref, h2_ref, wx_ref, wh1_ref, wh2_ref, wh3_ref,
                 bm1_ref, wm2_ref, bm2_ref, out_ref):
    dinv = _dinv(da_ref, db_ref)
    h3 = jnp.maximum(dinv * (aa_ref[...] + ab_ref[...] + g_ref[...]) + b_ref[...], 0.0)
    m = jnp.dot(x_ref[...], wx_ref[...], preferred_element_type=jnp.float32)
    m += jnp.dot(h1_ref[...], wh1_ref[...], preferred_element_type=jnp.float32)
    m += jnp.dot(h2_ref[...], wh2_ref[...], preferred_element_type=jnp.float32)
    m += jnp.dot(h3, wh3_ref[...], preferred_element_type=jnp.float32)
    m = jnp.maximum(m + bm1_ref[...], 0.0)
    z = jnp.dot(m, wm2_ref[...], preferred_element_type=jnp.float32) + bm2_ref[...]
    z -= jnp.max(z, axis=1, keepdims=True)
    ez = jnp.exp(z)
    out_ref[...] = ez / jnp.sum(ez, axis=1, keepdims=True)


def _rows(nc):
    return pl.BlockSpec((RB, nc), lambda i: (i, 0))


def _full(nr, nc):
    return pl.BlockSpec((nr, nc), lambda i: (0, 0))


_tc0 = pl.pallas_call(
    _tc0_body,
    grid=GRID,
    in_specs=[_rows(IN_CH), _full(IN_CH, HID), _rows(DEGW), _rows(DEGW)],
    out_specs=_rows(HID),
    out_shape=jax.ShapeDtypeStruct((N_NODES, HID), jnp.float32),
)

_tc_mid = pl.pallas_call(
    _tc_mid_body,
    grid=GRID,
    in_specs=[_rows(HID), _rows(HID), _rows(HID), _rows(DEGW), _rows(DEGW),
              _full(1, HID), _full(HID, HID)],
    out_specs=[_rows(HID), _rows(HID)],
    out_shape=[jax.ShapeDtypeStruct((N_NODES, HID), jnp.float32),
               jax.ShapeDtypeStruct((N_NODES, HID), jnp.float32)],
)

_tc_fin = pl.pallas_call(
    _tc_fin_body,
    grid=GRID,
    in_specs=[_rows(HID), _rows(HID), _rows(HID), _rows(DEGW), _rows(DEGW),
              _full(1, HID), _rows(IN_CH), _rows(HID), _rows(HID),
              _full(IN_CH, HID), _full(HID, HID), _full(HID, HID),
              _full(HID, HID), _full(1, HID), _full(HID, OUT_CH),
              _full(1, OUT_CH)],
    out_specs=_rows(OUT_CH),
    out_shape=jax.ShapeDtypeStruct((N_NODES, OUT_CH), jnp.float32),
)


@jax.jit
def kernel(x, edge_index, W1, b1, W2, b2, W3, b3, Wm1, bm1, Wm2, bm2):
    src = edge_index[0].astype(jnp.int32).reshape(NW, NCHUNK, CHUNK)
    dst = edge_index[1].astype(jnp.int32).reshape(NW, NCHUNK, CHUNK)

    ones16 = jnp.ones((CHUNK, DEGW), jnp.float32)
    zeros16 = jnp.zeros((ROW_BLK, DEGW), jnp.float32)
    zeros64 = jnp.zeros((ROW_BLK, HID), jnp.float32)

    deg = _sc_deg(dst, ones16, zeros16)
    da, db = deg[0], deg[1]

    g1 = _tc0(x, W1, da, db)
    acc1 = _sc_scatter(g1, src, dst, zeros64)
    h1, g2 = _tc_mid(acc1[0], acc1[1], g1, da, db, b1.reshape(1, HID), W2)
    acc2 = _sc_scatter(g2, src, dst, zeros64)
    h2, g3 = _tc_mid(acc2[0], acc2[1], g2, da, db, b2.reshape(1, HID), W3)
    acc3 = _sc_scatter(g3, src, dst, zeros64)

    return _tc_fin(acc3[0], acc3[1], g3, da, db, b3.reshape(1, HID),
                   x, h1, h2,
                   Wm1[:IN_CH], Wm1[IN_CH:IN_CH + HID],
                   Wm1[IN_CH + HID:IN_CH + 2 * HID], Wm1[IN_CH + 2 * HID:],
                   bm1.reshape(1, HID), Wm2, bm2.reshape(1, OUT_CH))


# speedup vs baseline: 26.2700x; 1.4903x over previous
import functools

import jax
import jax.numpy as jnp
from jax import lax
from jax.experimental import pallas as pl
from jax.experimental.pallas import tpu as pltpu
from jax.experimental.pallas import tpu_sc as plsc

N_NODES = 10000
N_EDGES = 320000
IN_CH = 128
HID = 64
OUT_CH = 64

NC, NS = 2, 16
NW = NC * NS
EPW = N_EDGES // NW
CHUNK = 100
NCHUNK = EPW // CHUNK
DEGW = 16
ROW_BLK = 80
NROWBLK = N_NODES // ROW_BLK

_MESH = plsc.VectorSubcoreMesh(core_axis_name="c", subcore_axis_name="s")


def _worker_ids():
    c = lax.axis_index("c")
    s = lax.axis_index("s")
    return c, s, c * NS + s


def _deg_body(dst_hbm, ones_hbm, zeros_hbm, out_hbm, dstv, onesv, zerosv, acc, sem):
    c, s, w = _worker_ids()
    pltpu.sync_copy(zeros_hbm, zerosv)

    @pl.loop(s, NROWBLK, step=NS)
    def _zero(k):
        pltpu.sync_copy(zerosv, acc.at[pl.ds(k * ROW_BLK, ROW_BLK)])

    plsc.subcore_barrier()
    pltpu.sync_copy(ones_hbm, onesv)
    pltpu.sync_copy(dst_hbm.at[w], dstv)

    @pl.loop(0, NCHUNK)
    def _edges(j):
        pltpu.sync_copy(onesv, acc.at[dstv.at[j]], add=True)

    plsc.subcore_barrier()

    @pl.loop(s, NROWBLK, step=NS)
    def _out(k):
        pltpu.sync_copy(acc.at[pl.ds(k * ROW_BLK, ROW_BLK)],
                        out_hbm.at[c, pl.ds(k * ROW_BLK, ROW_BLK)])


_sc_deg = pl.kernel(
    _deg_body,
    out_type=jax.ShapeDtypeStruct((NC, N_NODES, DEGW), jnp.float32),
    mesh=_MESH,
    scratch_types=[
        pltpu.VMEM((NCHUNK, CHUNK), jnp.int32),
        pltpu.VMEM((CHUNK, DEGW), jnp.float32),
        pltpu.VMEM((ROW_BLK, DEGW), jnp.float32),
        pltpu.VMEM_SHARED((N_NODES, DEGW), jnp.float32),
        pltpu.SemaphoreType.DMA,
    ],
    compiler_params=pltpu.CompilerParams(use_tc_tiling_on_sc=False),
)


def _scatter_body(g_hbm, src_hbm, dst_hbm, zeros_hbm, out_hbm,
                  srcv, dstv, rows0, rows1, zerosv, acc, sem0, sem1):
    c, s, w = _worker_ids()
    pltpu.sync_copy(zeros_hbm, zerosv)

    @pl.loop(s, NROWBLK, step=NS)
    def _zero(k):
        pltpu.sync_copy(zerosv, acc.at[pl.ds(k * ROW_BLK, ROW_BLK)])

    plsc.subcore_barrier()
    pltpu.sync_copy(src_hbm.at[w], srcv)
    pltpu.sync_copy(dst_hbm.at[w], dstv)

    pltpu.async_copy(g_hbm.at[srcv.at[0]], rows0, sem0)

    @pl.loop(0, NCHUNK, step=2)
    def _edges(j):
        cp1 = pltpu.async_copy(g_hbm.at[srcv.at[j + 1]], rows1, sem1)
        pltpu.make_async_copy(g_hbm.at[srcv.at[j]], rows0, sem0).wait()
        pltpu.sync_copy(rows0, acc.at[dstv.at[j]], add=True)

        @pl.when(j + 2 < NCHUNK)
        def _prefetch():
            pltpu.async_copy(g_hbm.at[srcv.at[j + 2]], rows0, sem0)

        cp1.wait()
        pltpu.sync_copy(rows1, acc.at[dstv.at[j + 1]], add=True)

    plsc.subcore_barrier()

    @pl.loop(s, NROWBLK, step=NS)
    def _out(k):
        pltpu.sync_copy(acc.at[pl.ds(k * ROW_BLK, ROW_BLK)],
                        out_hbm.at[c, pl.ds(k * ROW_BLK, ROW_BLK)])


_sc_scatter = pl.kernel(
    _scatter_body,
    out_type=jax.ShapeDtypeStruct((NC, N_NODES, HID), jnp.float32),
    mesh=_MESH,
    scratch_types=[
        pltpu.VMEM((NCHUNK, CHUNK), jnp.int32),
        pltpu.VMEM((NCHUNK, CHUNK), jnp.int32),
        pltpu.VMEM((CHUNK, HID), jnp.float32),
        pltpu.VMEM((CHUNK, HID), jnp.float32),
        pltpu.VMEM((ROW_BLK, HID), jnp.float32),
        pltpu.VMEM_SHARED((N_NODES, HID), jnp.float32),
        pltpu.SemaphoreType.DMA,
        pltpu.SemaphoreType.DMA,
    ],
    compiler_params=pltpu.CompilerParams(use_tc_tiling_on_sc=False),
)


RB = 1000
GRID = (N_NODES // RB,)


def _dinv(da_ref, db_ref):
    return lax.rsqrt(1.0 + da_ref[:, 0:1] + db_ref[:, 0:1])


def _tc0_body(x_ref, w_ref, da_ref, db_ref, g_ref):
    g_ref[...] = _dinv(da_ref, db_ref) * jnp.dot(
        x_ref[...], w_ref[...], preferred_element_type=jnp.float32)


def _tc_mid_body(aa_ref, ab_ref, g_ref, da_ref, db_ref, b_ref, w_ref,
                 h_ref, g2_ref):
    dinv = _dinv(da_ref, db_ref)
    h = jnp.maximum(dinv * (aa_ref[...] + ab_ref[...] + g_ref[...]) + b_ref[...], 0.0)
    h_ref[...] = h
    g2_ref[...] = dinv * jnp.dot(h, w_ref[...], preferred_element_type=jnp.float32)


def _tc_fin_body(aa_ref, ab_ref, g_ref, da_ref, db_ref, b_ref,
                 x_ref, h1_ref, h2_ref, wx_ref, wh1_ref, wh2_ref, wh3_ref,
                 bm1_ref, wm2_ref, bm2_ref, out_ref):
    dinv = _dinv(da_ref, db_ref)
    h3 = jnp.maximum(dinv * (aa_ref[...] + ab_ref[...] + g_ref[...]) + b_ref[...], 0.0)
    m = jnp.dot(x_ref[...], wx_ref[...], preferred_element_type=jnp.float32)
    m += jnp.dot(h1_ref[...], wh1_ref[...], preferred_element_type=jnp.float32)
    m += jnp.dot(h2_ref[...], wh2_ref[...], preferred_element_type=jnp.float32)
    m += jnp.dot(h3, wh3_ref[...], preferred_element_type=jnp.float32)
    m = jnp.maximum(m + bm1_ref[...], 0.0)
    z = jnp.dot(m, wm2_ref[...], preferred_element_type=jnp.float32) + bm2_ref[...]
    z -= jnp.max(z, axis=1, keepdims=True)
    ez = jnp.exp(z)
    out_ref[...] = ez / jnp.sum(ez, axis=1, keepdims=True)


def _rows(nc):
    return pl.BlockSpec((RB, nc), lambda i: (i, 0))


def _full(nr, nc):
    return pl.BlockSpec((nr, nc), lambda i: (0, 0))


_tc0 = pl.pallas_call(
    _tc0_body,
    grid=GRID,
    in_specs=[_rows(IN_CH), _full(IN_CH, HID), _rows(DEGW), _rows(DEGW)],
    out_specs=_rows(HID),
    out_shape=jax.ShapeDtypeStruct((N_NODES, HID), jnp.float32),
)

_tc_mid = pl.pallas_call(
    _tc_mid_body,
    grid=GRID,
    in_specs=[_rows(HID), _rows(HID), _rows(HID), _rows(DEGW), _rows(DEGW),
              _full(1, HID), _full(HID, HID)],
    out_specs=[_rows(HID), _rows(HID)],
    out_shape=[jax.ShapeDtypeStruct((N_NODES, HID), jnp.float32),
               jax.ShapeDtypeStruct((N_NODES, HID), jnp.float32)],
)

_tc_fin = pl.pallas_call(
    _tc_fin_body,
    grid=GRID,
    in_specs=[_rows(HID), _rows(HID), _rows(HID), _rows(DEGW), _rows(DEGW),
              _full(1, HID), _rows(IN_CH), _rows(HID), _rows(HID),
              _full(IN_CH, HID), _full(HID, HID), _full(HID, HID),
              _full(HID, HID), _full(1, HID), _full(HID, OUT_CH),
              _full(1, OUT_CH)],
    out_specs=_rows(OUT_CH),
    out_shape=jax.ShapeDtypeStruct((N_NODES, OUT_CH), jnp.float32),
)


@jax.jit
def kernel(x, edge_index, W1, b1, W2, b2, W3, b3, Wm1, bm1, Wm2, bm2):
    src = edge_index[0].astype(jnp.int32).reshape(NW, NCHUNK, CHUNK)
    dst = edge_index[1].astype(jnp.int32).reshape(NW, NCHUNK, CHUNK)

    ones16 = jnp.ones((CHUNK, DEGW), jnp.float32)
    zeros16 = jnp.zeros((ROW_BLK, DEGW), jnp.float32)
    zeros64 = jnp.zeros((ROW_BLK, HID), jnp.float32)

    deg = _sc_deg(dst, ones16, zeros16)
    da, db = deg[0], deg[1]

    g1 = _tc0(x, W1, da, db)
    acc1 = _sc_scatter(g1, src, dst, zeros64)
    h1, g2 = _tc_mid(acc1[0], acc1[1], g1, da, db, b1.reshape(1, HID), W2)
    acc2 = _sc_scatter(g2, src, dst, zeros64)
    h2, g3 = _tc_mid(acc2[0], acc2[1], g2, da, db, b2.reshape(1, HID), W3)
    acc3 = _sc_scatter(g3, src, dst, zeros64)

    return _tc_fin(acc3[0], acc3[1], g3, da, db, b3.reshape(1, HID),
                   x, h1, h2,
                   Wm1[:IN_CH], Wm1[IN_CH:IN_CH + HID],
                   Wm1[IN_CH + HID:IN_CH + 2 * HID], Wm1[IN_CH + 2 * HID:],
                   bm1.reshape(1, HID), Wm2, bm2.reshape(1, OUT_CH))
